# baseline (speedup 1.0000x reference)
import functools

import jax
import jax.numpy as jnp
from jax import lax
from jax.experimental import pallas as pl
from jax.experimental.pallas import tpu as pltpu
from jax.experimental.pallas import tpu_sc as plsc

GRID_ = 48
BATCH_ = 2
CIN_ = 16
COUT_ = 32
NPTS = 80000
EPS_ = 1e-5

P_ = GRID_ + 2
PLANE_ = P_ * P_
VOL_ = P_ * PLANE_

NC_ = 2
NS_ = 16
NW_ = NC_ * NS_

SROWS_ = 7816
SLAST_ = VOL_ - (NS_ - 1) * SROWS_
SPAD_ = NS_ * SROWS_
DUMP_ = VOL_
NPAD_ = 81920
PT_PTS_ = NPAD_ // NS_
SCH_ = 128
NSCH_ = PT_PTS_ // SCH_
SIG_ = 2
SIGCH_ = NSCH_ // SIG_

GPTS_ = NPAD_ // NW_
GCH_ = 128
NGCH_ = GPTS_ // GCH_

CROWS_ = 2398

@functools.cache
def _mesh():
    return plsc.VectorSubcoreMesh(core_axis_name="c", subcore_axis_name="s",
                                  num_cores=NC_, num_subcores=NS_)


def _scatter_body(feat_hbm, idx_hbm, zeros_hbm, out_hbm, acc_sh, featv, idxv):
    c = lax.axis_index("c")
    s = lax.axis_index("s")

    pltpu.sync_copy(zeros_hbm.at[pl.ds(s * SROWS_, SROWS_)],
                    acc_sh.at[pl.ds(s * SROWS_, SROWS_)])
    plsc.subcore_barrier()

    def _outer(h, _):
        pltpu.sync_copy(idx_hbm.at[c, s, h], idxv)

        def _inner(j, _):
            base = s * PT_PTS_ + h * (SIGCH_ * SCH_) + j * SCH_
            pltpu.sync_copy(feat_hbm.at[pl.ds(base, SCH_)], featv)
            pltpu.sync_copy(featv, acc_sh.at[idxv.at[j]], add=True)
            return 0
        lax.fori_loop(0, SIGCH_, _inner, 0)
        return 0
    lax.fori_loop(0, SIG_, _outer, 0)
    plsc.subcore_barrier()

    @pl.when(s < NS_ - 1)
    def _full():
        pltpu.sync_copy(acc_sh.at[pl.ds(s * SROWS_, SROWS_)],
                        out_hbm.at[c, pl.ds(s * SROWS_, SROWS_)])

    @pl.when(s == NS_ - 1)
    def _last():
        pltpu.sync_copy(acc_sh.at[pl.ds((NS_ - 1) * SROWS_, SLAST_)],
                        out_hbm.at[c, pl.ds((NS_ - 1) * SROWS_, SLAST_)])


@functools.cache
def _scatter():
    return pl.kernel(
        _scatter_body,
        out_type=jax.ShapeDtypeStruct((BATCH_, VOL_, CIN_), jnp.float32),
        mesh=_mesh(),
        compiler_params=pltpu.CompilerParams(use_tc_tiling_on_sc=False),
        scratch_types=[
            pltpu.VMEM_SHARED((SPAD_, CIN_), jnp.float32),
            pltpu.VMEM((SCH_, CIN_), jnp.float32),
            pltpu.VMEM((SIGCH_, SCH_), jnp.int32),
        ],
    )


def _conv_body(x0_ref, x1_ref, x2_ref, w_ref, b_ref, o_ref):
    planes = (x0_ref, x1_ref, x2_ref)
    pieces = []
    for dz in range(3):
        for dy in range(3):
            for dx in range(3):
                sh = (dy - 1) * P_ + (dx - 1)
                pieces.append(planes[dz][0, 0, pl.ds(51 + sh, CROWS_), :])
    xcat = jnp.concatenate(pieces, axis=1)
    acc = jnp.dot(xcat, w_ref[...], preferred_element_type=jnp.float32)
    acc = jnp.maximum(acc + b_ref[...], 0.0)
    o_ref[0, 0, pl.ds(51, CROWS_), :] = acc


def _conv(dense, w2, b2):
    grid = (BATCH_, GRID_)
    return pl.pallas_call(
        _conv_body,
        grid=grid,
        in_specs=[
            pl.BlockSpec((1, 1, PLANE_, CIN_), lambda b, z: (b, z, 0, 0)),
            pl.BlockSpec((1, 1, PLANE_, CIN_), lambda b, z: (b, z + 1, 0, 0)),
            pl.BlockSpec((1, 1, PLANE_, CIN_), lambda b, z: (b, z + 2, 0, 0)),
            pl.BlockSpec((27 * CIN_, COUT_), lambda b, z: (0, 0)),
            pl.BlockSpec((1, COUT_), lambda b, z: (0, 0)),
        ],
        out_specs=pl.BlockSpec((1, 1, PLANE_, COUT_), lambda b, z: (b, z + 1, 0, 0)),
        out_shape=jax.ShapeDtypeStruct((BATCH_, P_, PLANE_, COUT_), jnp.float32),
    )(dense, dense, dense, w2, b2)


def _gather_body(src_hbm, qidx_hbm, out_hbm, idxv, rows, sem):
    c = lax.axis_index("c")
    s = lax.axis_index("s")
    wid = c * NS_ + s
    pltpu.sync_copy(qidx_hbm.at[wid], idxv)
    copies = []
    for j in range(NGCH_):
        copies.append(pltpu.async_copy(
            src_hbm.at[idxv.at[j]], rows.at[pl.ds(j * GCH_, GCH_)], sem))
    for cp in copies:
        cp.wait()
    pltpu.sync_copy(rows, out_hbm.at[pl.ds(wid * GPTS_, GPTS_)])


@functools.cache
def _gather():
    return pl.kernel(
        _gather_body,
        out_type=jax.ShapeDtypeStruct((NPAD_, COUT_), jnp.float32),
        mesh=_mesh(),
        compiler_params=pltpu.CompilerParams(use_tc_tiling_on_sc=False),
        scratch_types=[
            pltpu.VMEM((NGCH_, GCH_), jnp.int32),
            pltpu.VMEM((GPTS_, COUT_), jnp.float32),
            pltpu.SemaphoreType.DMA,
        ],
    )


def _bn_body(x_ref, g_ref, bt_ref, o_ref):
    x = x_ref[...]
    nrows = NPTS // 4
    mask = lax.broadcasted_iota(jnp.int32, (NPAD_ // 4, 1), 0) < nrows
    xm = jnp.where(mask, x, 0.0)
    s1 = jnp.sum(xm, axis=0, keepdims=True)
    s2 = jnp.sum(xm * xm, axis=0, keepdims=True)
    s1 = (s1[:, 0:32] + s1[:, 32:64]) + (s1[:, 64:96] + s1[:, 96:128])
    s2 = (s2[:, 0:32] + s2[:, 32:64]) + (s2[:, 64:96] + s2[:, 96:128])
    mean = s1 / NPTS
    var = s2 / NPTS - mean * mean
    scale = lax.rsqrt(var + EPS_) * g_ref[...]
    shift = bt_ref[...] - mean * scale
    scale4 = jnp.concatenate([scale] * 4, axis=1)
    shift4 = jnp.concatenate([shift] * 4, axis=1)
    o_ref[...] = (x * scale4 + shift4)[: nrows, :]


def _bn(y0, gamma, beta):
    return pl.pallas_call(
        _bn_body,
        in_specs=[
            pl.BlockSpec((NPAD_ // 4, 128), lambda: (0, 0)),
            pl.BlockSpec((1, COUT_), lambda: (0, 0)),
            pl.BlockSpec((1, COUT_), lambda: (0, 0)),
        ],
        out_specs=pl.BlockSpec((NPTS // 4, 128), lambda: (0, 0)),
        out_shape=jax.ShapeDtypeStruct((NPTS // 4, 128), jnp.float32),
    )(y0, gamma, beta)


def kernel(features, coords, batch_idx, W, b, gamma, beta):
    f32 = jnp.float32
    pidx = ((coords[:, 0] + 1) * P_ + (coords[:, 1] + 1)) * P_ + (coords[:, 2] + 1)
    pidx = pidx.astype(jnp.int32)

    padn = NPAD_ - NPTS
    idx_c = jnp.stack([jnp.where(batch_idx == c, pidx, DUMP_) for c in range(BATCH_)])
    idx_c = jnp.pad(idx_c, ((0, 0), (0, padn)), constant_values=DUMP_)
    idx_c = idx_c.reshape(BATCH_, NS_, SIG_, SIGCH_, SCH_)
    feat_p = jnp.pad(features, ((0, padn), (0, 0)))
    zeros = jnp.zeros((SPAD_, CIN_), jnp.float32)

    dense = _scatter()(feat_p, idx_c, zeros)
    dense = dense.reshape(BATCH_, P_, PLANE_, CIN_)

    w2 = W.transpose(2, 3, 4, 1, 0).reshape(27 * CIN_, COUT_).astype(f32)
    b2 = b.reshape(1, COUT_).astype(f32)
    out_dense = _conv(dense, w2, b2)
    return out_dense[:, 1:17, :, :].reshape(NPTS, COUT_)

    qidx = (batch_idx.astype(jnp.int32) * VOL_ + pidx).astype(jnp.int32)
    qidx = jnp.pad(qidx, (0, padn), constant_values=2551)
    qidx = qidx.reshape(NW_, NGCH_, GCH_)
    flat = out_dense.reshape(BATCH_ * VOL_, COUT_)
    y0 = _gather()(flat, qidx)

    y = _bn(y0.reshape(NPAD_ // 4, 128), gamma.reshape(1, COUT_),
            beta.reshape(1, COUT_))
    return y.reshape(NPTS, COUT_)


# speedup vs baseline: 1.2249x; 1.2249x over previous
import functools

import jax
import jax.numpy as jnp
from jax import lax
from jax.experimental import pallas as pl
from jax.experimental.pallas import tpu as pltpu
from jax.experimental.pallas import tpu_sc as plsc

GRID_ = 48
BATCH_ = 2
CIN_ = 16
COUT_ = 32
NPTS = 80000
EPS_ = 1e-5

P_ = GRID_ + 2
PLANE_ = P_ * P_
VOL_ = P_ * PLANE_

NC_ = 2
NS_ = 16
NW_ = NC_ * NS_

SROWS_ = 7816
SLAST_ = VOL_ - (NS_ - 1) * SROWS_
SPAD_ = NS_ * SROWS_
DUMP_ = VOL_
NPAD_ = 81920
PT_PTS_ = NPAD_ // NS_
SCH_ = 128
NSCH_ = PT_PTS_ // SCH_
SIG_ = 2
SIGCH_ = NSCH_ // SIG_

GPTS_ = NPAD_ // NW_
GCH_ = 128
NGCH_ = GPTS_ // GCH_

CROWS_ = 2398

@functools.cache
def _mesh():
    return plsc.VectorSubcoreMesh(core_axis_name="c", subcore_axis_name="s",
                                  num_cores=NC_, num_subcores=NS_)


def _scatter_body(feat_hbm, idx_hbm, zeros_hbm, out_hbm, acc_sh, featv, idxv):
    c = lax.axis_index("c")
    s = lax.axis_index("s")

    pltpu.sync_copy(zeros_hbm.at[pl.ds(s * SROWS_, SROWS_)],
                    acc_sh.at[pl.ds(s * SROWS_, SROWS_)])
    plsc.subcore_barrier()

    def _outer(h, _):
        pltpu.sync_copy(idx_hbm.at[c, s, h], idxv)

        def _inner(j, _):
            base = s * PT_PTS_ + h * (SIGCH_ * SCH_) + j * SCH_
            pltpu.sync_copy(feat_hbm.at[pl.ds(base, SCH_)], featv)
            pltpu.sync_copy(featv, acc_sh.at[idxv.at[j]], add=True)
            return 0
        lax.fori_loop(0, SIGCH_, _inner, 0)
        return 0
    lax.fori_loop(0, SIG_, _outer, 0)
    plsc.subcore_barrier()

    @pl.when(s < NS_ - 1)
    def _full():
        pltpu.sync_copy(acc_sh.at[pl.ds(s * SROWS_, SROWS_)],
                        out_hbm.at[c, pl.ds(s * SROWS_, SROWS_)])

    @pl.when(s == NS_ - 1)
    def _last():
        pltpu.sync_copy(acc_sh.at[pl.ds((NS_ - 1) * SROWS_, SLAST_)],
                        out_hbm.at[c, pl.ds((NS_ - 1) * SROWS_, SLAST_)])


@functools.cache
def _scatter():
    return pl.kernel(
        _scatter_body,
        out_type=jax.ShapeDtypeStruct((BATCH_, VOL_, CIN_), jnp.float32),
        mesh=_mesh(),
        compiler_params=pltpu.CompilerParams(use_tc_tiling_on_sc=False),
        scratch_types=[
            pltpu.VMEM_SHARED((SPAD_, CIN_), jnp.float32),
            pltpu.VMEM((SCH_, CIN_), jnp.float32),
            pltpu.VMEM((SIGCH_, SCH_), jnp.int32),
        ],
    )


def _conv_body(x_ref, w_ref, b_ref, o_ref, g_ref):
    zp = pl.program_id(1)
    slot = lax.rem(zp, 3)
    xb = x_ref[0, 0].astype(jnp.bfloat16)
    xc = jnp.concatenate(
        [xb[0:PLANE_ - 2], xb[1:PLANE_ - 1], xb[2:PLANE_]], axis=1)
    g = jnp.dot(xc[0:CROWS_], w_ref[0], preferred_element_type=jnp.float32)
    g = g + jnp.dot(xc[50:50 + CROWS_], w_ref[1],
                    preferred_element_type=jnp.float32)
    g = g + jnp.dot(xc[100:100 + CROWS_], w_ref[2],
                    preferred_element_type=jnp.float32)
    g_ref[slot, 0:CROWS_, :] = g

    @pl.when(zp >= 2)
    def _emit():
        s0 = lax.rem(zp + 1, 3)
        s1 = lax.rem(zp + 2, 3)
        acc = (g_ref[s0, 0:CROWS_, 0:32] + g_ref[s1, 0:CROWS_, 32:64]
               + g_ref[slot, 0:CROWS_, 64:96])
        o_ref[0, 0, pl.ds(51, CROWS_), :] = jnp.maximum(acc + b_ref[...], 0.0)


def _conv(dense, w3, b2):
    grid = (BATCH_, P_)
    return pl.pallas_call(
        _conv_body,
        grid=grid,
        in_specs=[
            pl.BlockSpec((1, 1, PLANE_, CIN_), lambda b, z: (b, z, 0, 0)),
            pl.BlockSpec((3, 48, 96), lambda b, z: (0, 0, 0)),
            pl.BlockSpec((1, COUT_), lambda b, z: (0, 0)),
        ],
        out_specs=pl.BlockSpec((1, 1, PLANE_, COUT_),
                               lambda b, z: (b, jnp.maximum(z - 1, 0), 0, 0)),
        out_shape=jax.ShapeDtypeStruct((BATCH_, P_, PLANE_, COUT_), jnp.float32),
        scratch_shapes=[pltpu.VMEM((3, CROWS_, 96), jnp.float32)],
    )(dense, w3, b2)


def _gather_body(src_hbm, qidx_hbm, out_hbm, idxv, rows, sem):
    c = lax.axis_index("c")
    s = lax.axis_index("s")
    wid = c * NS_ + s
    pltpu.sync_copy(qidx_hbm.at[wid], idxv)
    copies = []
    for j in range(NGCH_):
        copies.append(pltpu.async_copy(
            src_hbm.at[idxv.at[j]], rows.at[pl.ds(j * GCH_, GCH_)], sem))
    for cp in copies:
        cp.wait()
    pltpu.sync_copy(rows, out_hbm.at[pl.ds(wid * GPTS_, GPTS_)])


@functools.cache
def _gather():
    return pl.kernel(
        _gather_body,
        out_type=jax.ShapeDtypeStruct((NPAD_, COUT_), jnp.float32),
        mesh=_mesh(),
        compiler_params=pltpu.CompilerParams(use_tc_tiling_on_sc=False),
        scratch_types=[
            pltpu.VMEM((NGCH_, GCH_), jnp.int32),
            pltpu.VMEM((GPTS_, COUT_), jnp.float32),
            pltpu.SemaphoreType.DMA,
        ],
    )


def _bn_body(x_ref, g_ref, bt_ref, o_ref):
    x = x_ref[...]
    nrows = NPTS // 4
    mask = lax.broadcasted_iota(jnp.int32, (NPAD_ // 4, 1), 0) < nrows
    xm = jnp.where(mask, x, 0.0)
    s1 = jnp.sum(xm, axis=0, keepdims=True)
    s2 = jnp.sum(xm * xm, axis=0, keepdims=True)
    s1 = (s1[:, 0:32] + s1[:, 32:64]) + (s1[:, 64:96] + s1[:, 96:128])
    s2 = (s2[:, 0:32] + s2[:, 32:64]) + (s2[:, 64:96] + s2[:, 96:128])
    mean = s1 / NPTS
    var = s2 / NPTS - mean * mean
    scale = lax.rsqrt(var + EPS_) * g_ref[...]
    shift = bt_ref[...] - mean * scale
    scale4 = jnp.concatenate([scale] * 4, axis=1)
    shift4 = jnp.concatenate([shift] * 4, axis=1)
    o_ref[...] = (x * scale4 + shift4)[: nrows, :]


def _bn(y0, gamma, beta):
    return pl.pallas_call(
        _bn_body,
        in_specs=[
            pl.BlockSpec((NPAD_ // 4, 128), lambda: (0, 0)),
            pl.BlockSpec((1, COUT_), lambda: (0, 0)),
            pl.BlockSpec((1, COUT_), lambda: (0, 0)),
        ],
        out_specs=pl.BlockSpec((NPTS // 4, 128), lambda: (0, 0)),
        out_shape=jax.ShapeDtypeStruct((NPTS // 4, 128), jnp.float32),
    )(y0, gamma, beta)


def kernel(features, coords, batch_idx, W, b, gamma, beta):
    f32 = jnp.float32
    pidx = ((coords[:, 0] + 1) * P_ + (coords[:, 1] + 1)) * P_ + (coords[:, 2] + 1)
    pidx = pidx.astype(jnp.int32)

    padn = NPAD_ - NPTS
    idx_c = jnp.stack([jnp.where(batch_idx == c, pidx, DUMP_) for c in range(BATCH_)])
    idx_c = jnp.pad(idx_c, ((0, 0), (0, padn)), constant_values=DUMP_)
    idx_c = idx_c.reshape(BATCH_, NS_, SIG_, SIGCH_, SCH_)
    feat_p = jnp.pad(features, ((0, padn), (0, 0)))
    zeros = jnp.zeros((SPAD_, CIN_), jnp.float32)

    dense = _scatter()(feat_p, idx_c, zeros)
    dense = dense.reshape(BATCH_, P_, PLANE_, CIN_)

    w3 = W.transpose(3, 4, 1, 2, 0).reshape(3, 48, 96).astype(jnp.bfloat16)
    b2 = b.reshape(1, COUT_).astype(f32)
    out_dense = _conv(dense, w3, b2)

    qidx = (batch_idx.astype(jnp.int32) * VOL_ + pidx).astype(jnp.int32)
    qidx = jnp.pad(qidx, (0, padn), constant_values=2551)
    qidx = qidx.reshape(NW_, NGCH_, GCH_)
    flat = out_dense.reshape(BATCH_ * VOL_, COUT_)
    y0 = _gather()(flat, qidx)

    y = _bn(y0.reshape(NPAD_ // 4, 128), gamma.reshape(1, COUT_),
            beta.reshape(1, COUT_))
    return y.reshape(NPTS, COUT_)


# speedup vs baseline: 1.2277x; 1.0023x over previous
import functools

import jax
import jax.numpy as jnp
from jax import lax
from jax.experimental import pallas as pl
from jax.experimental.pallas import tpu as pltpu
from jax.experimental.pallas import tpu_sc as plsc

GRID_ = 48
BATCH_ = 2
CIN_ = 16
COUT_ = 32
NPTS = 80000
EPS_ = 1e-5

P_ = GRID_ + 2
PLANE_ = P_ * P_
VOL_ = P_ * PLANE_

NC_ = 2
NS_ = 16
NW_ = NC_ * NS_

SROWS_ = 7816
SLAST_ = VOL_ - (NS_ - 1) * SROWS_
SPAD_ = NS_ * SROWS_
DUMP_ = VOL_
NPAD_ = 81920
PT_PTS_ = NPAD_ // NS_
SCH_ = 128
NSCH_ = PT_PTS_ // SCH_
SIG_ = 4
SIGCH_ = NSCH_ // SIG_

GPTS_ = NPAD_ // NW_
GCH_ = 128
NGCH_ = GPTS_ // GCH_

CROWS_ = 2398

@functools.cache
def _mesh():
    return plsc.VectorSubcoreMesh(core_axis_name="c", subcore_axis_name="s",
                                  num_cores=NC_, num_subcores=NS_)


def _scatter_body(feat_hbm, idx_hbm, zeros_hbm, out_hbm, acc_sh, featv, idxv,
                  semf, sems):
    c = lax.axis_index("c")
    s = lax.axis_index("s")

    pltpu.sync_copy(zeros_hbm.at[pl.ds(s * SROWS_, SROWS_)],
                    acc_sh.at[pl.ds(s * SROWS_, SROWS_)])
    plsc.subcore_barrier()
    def _src(h, jj):
        return feat_hbm.at[pl.ds(s * PT_PTS_ + (h * SIGCH_ + jj) * SCH_, SCH_)]

    def _group(h, _):
        pltpu.sync_copy(idx_hbm.at[c, s, h], idxv)
        pltpu.async_copy(_src(h, 0), featv.at[0], semf)

        def _chunk(jj, _):
            bjj = lax.rem(jj, 2)
            bnx = lax.rem(jj + 1, 2)

            @pl.when(jj >= 1)
            def _ws():
                pltpu.make_async_copy(featv.at[bnx],
                                      acc_sh.at[idxv.at[jj - 1]], sems).wait()

            @pl.when(jj < SIGCH_ - 1)
            def _fn():
                pltpu.async_copy(_src(h, jj + 1), featv.at[bnx], semf)
            pltpu.make_async_copy(_src(h, jj), featv.at[bjj], semf).wait()
            pltpu.async_copy(featv.at[bjj], acc_sh.at[idxv.at[jj]], sems,
                             add=True)
            return 0
        lax.fori_loop(0, SIGCH_, _chunk, 0)
        pltpu.make_async_copy(featv.at[lax.rem(SIGCH_ - 1, 2)],
                              acc_sh.at[idxv.at[SIGCH_ - 1]], sems).wait()
        return 0
    lax.fori_loop(0, SIG_, _group, 0)
    plsc.subcore_barrier()

    @pl.when(s < NS_ - 1)
    def _full():
        pltpu.sync_copy(acc_sh.at[pl.ds(s * SROWS_, SROWS_)],
                        out_hbm.at[c, pl.ds(s * SROWS_, SROWS_)])

    @pl.when(s == NS_ - 1)
    def _last():
        pltpu.sync_copy(acc_sh.at[pl.ds((NS_ - 1) * SROWS_, SLAST_)],
                        out_hbm.at[c, pl.ds((NS_ - 1) * SROWS_, SLAST_)])


@functools.cache
def _scatter():
    return pl.kernel(
        _scatter_body,
        out_type=jax.ShapeDtypeStruct((BATCH_, VOL_, CIN_), jnp.float32),
        mesh=_mesh(),
        compiler_params=pltpu.CompilerParams(use_tc_tiling_on_sc=False),
        scratch_types=[
            pltpu.VMEM_SHARED((SPAD_, CIN_), jnp.float32),
            pltpu.VMEM((2, SCH_, CIN_), jnp.float32),
            pltpu.VMEM((SIGCH_, SCH_), jnp.int32),
            pltpu.SemaphoreType.DMA,
            pltpu.SemaphoreType.DMA,
        ],
    )


def _conv_body(x_ref, w_ref, b_ref, o_ref, g_ref):
    zp = pl.program_id(1)
    slot = lax.rem(zp, 3)
    xb = x_ref[0, 0].astype(jnp.bfloat16)
    xc = jnp.concatenate(
        [xb[0:PLANE_ - 2], xb[1:PLANE_ - 1], xb[2:PLANE_]], axis=1)
    g = jnp.dot(xc[0:CROWS_], w_ref[0], preferred_element_type=jnp.float32)
    g = g + jnp.dot(xc[50:50 + CROWS_], w_ref[1],
                    preferred_element_type=jnp.float32)
    g = g + jnp.dot(xc[100:100 + CROWS_], w_ref[2],
                    preferred_element_type=jnp.float32)
    g_ref[slot, 0:CROWS_, :] = g

    @pl.when(zp >= 2)
    def _emit():
        s0 = lax.rem(zp + 1, 3)
        s1 = lax.rem(zp + 2, 3)
        acc = (g_ref[s0, 0:CROWS_, 0:32] + g_ref[s1, 0:CROWS_, 32:64]
               + g_ref[slot, 0:CROWS_, 64:96])
        o_ref[0, 0, pl.ds(51, CROWS_), :] = jnp.maximum(acc + b_ref[...], 0.0)


def _conv(dense, w3, b2):
    grid = (BATCH_, P_)
    return pl.pallas_call(
        _conv_body,
        grid=grid,
        in_specs=[
            pl.BlockSpec((1, 1, PLANE_, CIN_), lambda b, z: (b, z, 0, 0)),
            pl.BlockSpec((3, 48, 96), lambda b, z: (0, 0, 0)),
            pl.BlockSpec((1, COUT_), lambda b, z: (0, 0)),
        ],
        out_specs=pl.BlockSpec((1, 1, PLANE_, COUT_),
                               lambda b, z: (b, jnp.maximum(z - 1, 0), 0, 0)),
        out_shape=jax.ShapeDtypeStruct((BATCH_, P_, PLANE_, COUT_), jnp.float32),
        scratch_shapes=[pltpu.VMEM((3, CROWS_, 96), jnp.float32)],
    )(dense, w3, b2)


def _gather_body(src_hbm, qidx_hbm, out_hbm, idxv, rows, sem):
    c = lax.axis_index("c")
    s = lax.axis_index("s")
    wid = c * NS_ + s
    pltpu.sync_copy(qidx_hbm.at[wid], idxv)
    copies = []
    for j in range(NGCH_):
        copies.append(pltpu.async_copy(
            src_hbm.at[idxv.at[j]], rows.at[pl.ds(j * GCH_, GCH_)], sem))
    for cp in copies:
        cp.wait()
    pltpu.sync_copy(rows, out_hbm.at[pl.ds(wid * GPTS_, GPTS_)])


@functools.cache
def _gather():
    return pl.kernel(
        _gather_body,
        out_type=jax.ShapeDtypeStruct((NPAD_, COUT_), jnp.float32),
        mesh=_mesh(),
        compiler_params=pltpu.CompilerParams(use_tc_tiling_on_sc=False),
        scratch_types=[
            pltpu.VMEM((NGCH_, GCH_), jnp.int32),
            pltpu.VMEM((GPTS_, COUT_), jnp.float32),
            pltpu.SemaphoreType.DMA,
        ],
    )


def _bn_body(x_ref, g_ref, bt_ref, o_ref):
    x = x_ref[...]
    nrows = NPTS // 4
    mask = lax.broadcasted_iota(jnp.int32, (NPAD_ // 4, 1), 0) < nrows
    xm = jnp.where(mask, x, 0.0)
    s1 = jnp.sum(xm, axis=0, keepdims=True)
    s2 = jnp.sum(xm * xm, axis=0, keepdims=True)
    s1 = (s1[:, 0:32] + s1[:, 32:64]) + (s1[:, 64:96] + s1[:, 96:128])
    s2 = (s2[:, 0:32] + s2[:, 32:64]) + (s2[:, 64:96] + s2[:, 96:128])
    mean = s1 / NPTS
    var = s2 / NPTS - mean * mean
    scale = lax.rsqrt(var + EPS_) * g_ref[...]
    shift = bt_ref[...] - mean * scale
    scale4 = jnp.concatenate([scale] * 4, axis=1)
    shift4 = jnp.concatenate([shift] * 4, axis=1)
    o_ref[...] = (x * scale4 + shift4)[: nrows, :]


def _bn(y0, gamma, beta):
    return pl.pallas_call(
        _bn_body,
        in_specs=[
            pl.BlockSpec((NPAD_ // 4, 128), lambda: (0, 0)),
            pl.BlockSpec((1, COUT_), lambda: (0, 0)),
            pl.BlockSpec((1, COUT_), lambda: (0, 0)),
        ],
        out_specs=pl.BlockSpec((NPTS // 4, 128), lambda: (0, 0)),
        out_shape=jax.ShapeDtypeStruct((NPTS // 4, 128), jnp.float32),
    )(y0, gamma, beta)


def kernel(features, coords, batch_idx, W, b, gamma, beta):
    f32 = jnp.float32
    pidx = ((coords[:, 0] + 1) * P_ + (coords[:, 1] + 1)) * P_ + (coords[:, 2] + 1)
    pidx = pidx.astype(jnp.int32)

    padn = NPAD_ - NPTS
    idx_c = jnp.stack([jnp.where(batch_idx == c, pidx, DUMP_) for c in range(BATCH_)])
    idx_c = jnp.pad(idx_c, ((0, 0), (0, padn)), constant_values=DUMP_)
    idx_c = idx_c.reshape(BATCH_, NS_, SIG_, SIGCH_, SCH_)
    feat_p = jnp.pad(features, ((0, padn), (0, 0)))
    zeros = jnp.zeros((SPAD_, CIN_), jnp.float32)

    dense = _scatter()(feat_p, idx_c, zeros)
    dense = dense.reshape(BATCH_, P_, PLANE_, CIN_)

    w3 = W.transpose(3, 4, 1, 2, 0).reshape(3, 48, 96).astype(jnp.bfloat16)
    b2 = b.reshape(1, COUT_).astype(f32)
    out_dense = _conv(dense, w3, b2)

    qidx = (batch_idx.astype(jnp.int32) * VOL_ + pidx).astype(jnp.int32)
    qidx = jnp.pad(qidx, (0, padn), constant_values=2551)
    qidx = qidx.reshape(NW_, NGCH_, GCH_)
    flat = out_dense.reshape(BATCH_ * VOL_, COUT_)
    y0 = _gather()(flat, qidx)

    y = _bn(y0.reshape(NPAD_ // 4, 128), gamma.reshape(1, COUT_),
            beta.reshape(1, COUT_))
    return y.reshape(NPTS, COUT_)


# speedup vs baseline: 1.6389x; 1.3349x over previous
import functools

import jax
import jax.numpy as jnp
from jax import lax
from jax.experimental import pallas as pl
from jax.experimental.pallas import tpu as pltpu
from jax.experimental.pallas import tpu_sc as plsc

GRID_ = 48
BATCH_ = 2
CIN_ = 16
COUT_ = 32
NPTS = 80000
EPS_ = 1e-5

PROWS_ = 2560
PPK_ = PROWS_ // 8
VOL_ = GRID_ * PROWS_
DUMP_ = VOL_
AROWS_ = VOL_ + 8

NC_ = 2
NS_ = 16
NW_ = NC_ * NS_

SROWS_ = VOL_ // NS_
NPAD_ = 81920
PT_PTS_ = NPAD_ // NS_
SCH_ = 128
NSCH_ = PT_PTS_ // SCH_
SIG_ = 4
SIGCH_ = NSCH_ // SIG_

GPTS_ = NPAD_ // NW_
GCH_ = 128
NGCH_ = GPTS_ // GCH_
GSRC_ = BATCH_ * GRID_ * 2 * PPK_ * 4

SHIFTS_ = [(dy - 1) * 50 + (dx - 1) for dy in range(3) for dx in range(3)]


@functools.cache
def _mesh():
    return plsc.VectorSubcoreMesh(core_axis_name="c", subcore_axis_name="s",
                                  num_cores=NC_, num_subcores=NS_)


def _scatter_body(feat_hbm, idx_hbm, zeros_hbm, out_hbm, acc_sh, featv, idxv,
                  semf, sems):
    c = lax.axis_index("c")
    s = lax.axis_index("s")

    pltpu.sync_copy(zeros_hbm.at[pl.ds(s * SROWS_, SROWS_)],
                    acc_sh.at[pl.ds(s * SROWS_, SROWS_)])

    @pl.when(s == 0)
    def _zdump():
        pltpu.sync_copy(zeros_hbm.at[pl.ds(0, AROWS_ - VOL_)],
                        acc_sh.at[pl.ds(VOL_, AROWS_ - VOL_)])
    plsc.subcore_barrier()

    def _src(h, jj):
        return feat_hbm.at[pl.ds(s * PT_PTS_ + (h * SIGCH_ + jj) * SCH_, SCH_)]

    def _group(h, _):
        pltpu.sync_copy(idx_hbm.at[c, s, h], idxv)
        pltpu.async_copy(_src(h, 0), featv.at[0], semf)

        def _chunk(jj, _):
            bjj = lax.rem(jj, 2)
            bnx = lax.rem(jj + 1, 2)

            @pl.when(jj >= 1)
            def _ws():
                pltpu.make_async_copy(featv.at[bnx],
                                      acc_sh.at[idxv.at[jj - 1]], sems).wait()

            @pl.when(jj < SIGCH_ - 1)
            def _fn():
                pltpu.async_copy(_src(h, jj + 1), featv.at[bnx], semf)
            pltpu.make_async_copy(_src(h, jj), featv.at[bjj], semf).wait()
            pltpu.async_copy(featv.at[bjj], acc_sh.at[idxv.at[jj]], sems,
                             add=True)
            return 0
        lax.fori_loop(0, SIGCH_, _chunk, 0)
        pltpu.make_async_copy(featv.at[lax.rem(SIGCH_ - 1, 2)],
                              acc_sh.at[idxv.at[SIGCH_ - 1]], sems).wait()
        return 0
    lax.fori_loop(0, SIG_, _group, 0)
    plsc.subcore_barrier()

    pltpu.sync_copy(acc_sh.at[pl.ds(s * SROWS_, SROWS_)],
                    out_hbm.at[c, pl.ds(s * SROWS_, SROWS_)])


@functools.cache
def _scatter():
    return pl.kernel(
        _scatter_body,
        out_type=jax.ShapeDtypeStruct((BATCH_, VOL_, CIN_), jnp.float32),
        mesh=_mesh(),
        compiler_params=pltpu.CompilerParams(use_tc_tiling_on_sc=False),
        scratch_types=[
            pltpu.VMEM_SHARED((AROWS_, CIN_), jnp.float32),
            pltpu.VMEM((2, SCH_, CIN_), jnp.float32),
            pltpu.VMEM((SIGCH_, SCH_), jnp.int32),
            pltpu.SemaphoreType.DMA,
            pltpu.SemaphoreType.DMA,
        ],
    )


def _shift16(x, s):
    m = s * 16
    a = m // 128
    b = m - a * 128

    def rr(k):
        if k == 0:
            return x
        if k > 0:
            return jnp.concatenate(
                [x[k:PPK_], jnp.zeros((k, 128), x.dtype)], axis=0)
        return jnp.concatenate(
            [jnp.zeros((-k, 128), x.dtype), x[0:PPK_ + k]], axis=0)
    va = rr(a)
    if b == 0:
        return va
    vb = rr(a + 1)
    return jnp.concatenate([va[:, b:], vb[:, :b]], axis=1)


def _conv_body(x_ref, w_ref, b_ref, o_ref, g_ref):
    zp = pl.program_id(1)
    slot = lax.rem(zp, 3)

    @pl.when(zp < GRID_)
    def _compute():
        xb = x_ref[0, 0].astype(jnp.bfloat16)
        g = jnp.zeros((PPK_, 768), jnp.float32)
        for k, s in enumerate(SHIFTS_):
            g = g + jnp.dot(_shift16(xb, s), w_ref[k],
                            preferred_element_type=jnp.float32)
        g_ref[slot] = g

    @pl.when(zp == 0)
    def _z2():
        g_ref[2] = jnp.zeros((PPK_, 768), jnp.float32)

    @pl.when(zp == GRID_)
    def _z0():
        g_ref[0] = jnp.zeros((PPK_, 768), jnp.float32)

    @pl.when(zp >= 1)
    def _emit():
        s0 = lax.rem(zp + 1, 3)
        s1 = lax.rem(zp + 2, 3)
        for par in range(2):
            acc = None
            for dz, sl in ((0, s0), (1, s1), (2, slot)):
                q = g_ref[sl]
                piece = jnp.concatenate(
                    [q[:, par * 384 + t4 * 96 + dz * 32:
                         par * 384 + t4 * 96 + dz * 32 + 32]
                     for t4 in range(4)], axis=1)
                acc = piece if acc is None else acc + piece
            acc = jnp.maximum(acc + b_ref[...], 0.0)
            o_ref[0, 0, par] = acc


def _conv(dense, wbd, b4):
    grid = (BATCH_, GRID_ + 1)
    return pl.pallas_call(
        _conv_body,
        grid=grid,
        in_specs=[
            pl.BlockSpec((1, 1, PPK_, 128),
                         lambda b, z: (b, jnp.minimum(z, GRID_ - 1), 0, 0)),
            pl.BlockSpec((9, 128, 768), lambda b, z: (0, 0, 0)),
            pl.BlockSpec((1, 128), lambda b, z: (0, 0)),
        ],
        out_specs=pl.BlockSpec((1, 1, 2, PPK_, 128),
                               lambda b, z: (b, jnp.maximum(z - 1, 0), 0, 0, 0)),
        out_shape=jax.ShapeDtypeStruct((BATCH_, GRID_, 2, PPK_, 128),
                                       jnp.float32),
        scratch_shapes=[pltpu.VMEM((3, PPK_, 768), jnp.float32)],
    )(dense, wbd, b4)


def _gather_body(src_hbm, qidx_hbm, out_hbm, idxv, rows, sem):
    c = lax.axis_index("c")
    s = lax.axis_index("s")
    wid = c * NS_ + s
    pltpu.sync_copy(qidx_hbm.at[wid], idxv)
    copies = []
    for j in range(NGCH_):
        copies.append(pltpu.async_copy(
            src_hbm.at[idxv.at[j]], rows.at[pl.ds(j * GCH_, GCH_)], sem))
    for cp in copies:
        cp.wait()
    pltpu.sync_copy(rows, out_hbm.at[pl.ds(wid * GPTS_, GPTS_)])


@functools.cache
def _gather():
    return pl.kernel(
        _gather_body,
        out_type=jax.ShapeDtypeStruct((NPAD_, COUT_), jnp.float32),
        mesh=_mesh(),
        compiler_params=pltpu.CompilerParams(use_tc_tiling_on_sc=False),
        scratch_types=[
            pltpu.VMEM((NGCH_, GCH_), jnp.int32),
            pltpu.VMEM((GPTS_, COUT_), jnp.float32),
            pltpu.SemaphoreType.DMA,
        ],
    )


def _bn_body(x_ref, g_ref, bt_ref, o_ref):
    x = x_ref[...]
    nrows = NPTS // 4
    mask = lax.broadcasted_iota(jnp.int32, (NPAD_ // 4, 1), 0) < nrows
    xm = jnp.where(mask, x, 0.0)
    s1 = jnp.sum(xm, axis=0, keepdims=True)
    s2 = jnp.sum(xm * xm, axis=0, keepdims=True)
    s1 = (s1[:, 0:32] + s1[:, 32:64]) + (s1[:, 64:96] + s1[:, 96:128])
    s2 = (s2[:, 0:32] + s2[:, 32:64]) + (s2[:, 64:96] + s2[:, 96:128])
    mean = s1 / NPTS
    var = s2 / NPTS - mean * mean
    scale = lax.rsqrt(var + EPS_) * g_ref[...]
    shift = bt_ref[...] - mean * scale
    scale4 = jnp.concatenate([scale] * 4, axis=1)
    shift4 = jnp.concatenate([shift] * 4, axis=1)
    o_ref[...] = (x * scale4 + shift4)[: nrows, :]


def _bn(y0, gamma, beta):
    return pl.pallas_call(
        _bn_body,
        in_specs=[
            pl.BlockSpec((NPAD_ // 4, 128), lambda: (0, 0)),
            pl.BlockSpec((1, COUT_), lambda: (0, 0)),
            pl.BlockSpec((1, COUT_), lambda: (0, 0)),
        ],
        out_specs=pl.BlockSpec((NPTS // 4, 128), lambda: (0, 0)),
        out_shape=jax.ShapeDtypeStruct((NPTS // 4, 128), jnp.float32),
    )(y0, gamma, beta)


def kernel(features, coords, batch_idx, W, b, gamma, beta):
    f32 = jnp.float32
    pidx = (coords[:, 0] * PROWS_ + (coords[:, 1] + 1) * 50
            + (coords[:, 2] + 1)).astype(jnp.int32)

    padn = NPAD_ - NPTS
    idx_c = jnp.stack([jnp.where(batch_idx == c, pidx, DUMP_)
                       for c in range(BATCH_)])
    idx_c = jnp.pad(idx_c, ((0, 0), (0, padn)), constant_values=DUMP_)
    idx_c = idx_c.reshape(BATCH_, NS_, SIG_, SIGCH_, SCH_)
    feat_p = jnp.pad(features, ((0, padn), (0, 0)))
    zeros = jnp.zeros((SROWS_, CIN_), f32)

    dense = _scatter()(feat_p, idx_c, zeros)
    dense = dense.reshape(BATCH_, GRID_, PPK_, 128)

    wk = W.transpose(3, 4, 1, 2, 0).reshape(9, CIN_, 3 * COUT_)
    eye8 = jnp.eye(8, dtype=f32)
    wbd = jnp.einsum("tu,kcj->ktcuj", eye8, wk).reshape(9, 128, 8 * 96)
    wbd = wbd.astype(jnp.bfloat16)
    b4 = jnp.tile(b.reshape(1, COUT_), (1, 4)).astype(f32)

    out_dense = _conv(dense, wbd, b4)

    prow = (coords[:, 1] + 1) * 50 + (coords[:, 2] + 1)
    p8, t = prow // 8, prow % 8
    qidx = ((((batch_idx * GRID_ + coords[:, 0]) * 2 + t // 4) * PPK_ + p8) * 4
            + t % 4).astype(jnp.int32)
    qidx = jnp.pad(qidx, (0, padn))
    qidx = qidx.reshape(NW_, NGCH_, GCH_)
    flat = out_dense.reshape(GSRC_, COUT_)
    y0 = _gather()(flat, qidx)

    y = _bn(y0.reshape(NPAD_ // 4, 128), gamma.reshape(1, COUT_),
            beta.reshape(1, COUT_))
    return y.reshape(NPTS, COUT_)
